# baseline (speedup 1.0000x reference)
import functools

import jax
import jax.numpy as jnp
from jax import lax
from jax.experimental import pallas as pl
from jax.experimental.pallas import tpu as pltpu
from jax.experimental.pallas import tpu_sc as plsc

NUM_CORES = 2
NUM_SUBCORES = 16
NUM_WORKERS = NUM_CORES * NUM_SUBCORES
LANES = 16

CH = 56
NCMAX = 9
SEGROWS = CH * NCMAX


def _sc_body(N, F, B, seq_h, meta_h, wlh_h, pvec_h, wsp_h, out_h,
             buf_v, m0_v, m1_v, wl_v, pv_v, wv_v, eff_v, row_v,
             sem_a, sem_b):
    nj = F // LANES
    CHW = CH * F
    BASE1 = SEGROWS * F

    cid = lax.axis_index("c")
    sid = lax.axis_index("s")
    wid = sid * NUM_CORES + cid

    pltpu.sync_copy(pvec_h, pv_v)
    pltpu.sync_copy(wsp_h, wv_v)
    pltpu.sync_copy(wlh_h.at[pl.ds(wid * 16, LANES)], wl_v)

    wl0 = wv_v[pl.ds(0, LANES)]
    wl1 = wv_v[pl.ds(LANES, LANES)]
    wl2 = wv_v[pl.ds(2 * LANES, LANES)]
    wd0 = wv_v[pl.ds(3 * LANES, LANES)]
    wd1 = wv_v[pl.ds(4 * LANES, LANES)]
    for j in range(nj):
        p1c = pv_v[pl.ds(j * LANES, LANES)]
        p2c = pv_v[pl.ds(F + j * LANES, LANES)]
        p3c = pv_v[pl.ds(2 * F + j * LANES, LANES)]
        wdc = pv_v[pl.ds(3 * F + j * LANES, LANES)]
        eff_v[pl.ds(j * LANES, LANES)] = (
            wd0 * (1.0 + wl0 * p1c + wl1 * p2c + wl2 * p3c) + wd1 * wdc)

    effs = tuple(eff_v[pl.ds(j * LANES, LANES)] for j in range(nj))

    wlv = wl_v[pl.ds(0, LANES)]
    lo = wlv[0]
    hi = wlv[1]

    def read_meta(k, mb):
        pltpu.sync_copy(meta_h.at[pl.ds(k * 16, LANES)], mb)

    def seg_params(mb):
        mv = mb[pl.ds(0, LANES)]
        return mv[0], mv[1]

    def fire(mb, base, sem):
        s, ln = seg_params(mb)
        nc = (ln + (CH - 1)) // CH
        for c in range(NCMAX):
            @pl.when(c < nc)
            def _(c=c):
                g = jnp.minimum(s + c * CH, N - CH)
                pltpu.async_copy(seq_h.at[pl.ds(g * F, CHW)],
                                 buf_v.at[pl.ds(base + c * CHW, CHW)], sem)

    def drain_compute_write(mb, base, sem, b):
        s, ln = seg_params(mb)
        nc = (ln + (CH - 1)) // CH
        for c in range(NCMAX):
            @pl.when(c < nc)
            def _(c=c):
                pltpu.make_async_copy(
                    seq_h.at[pl.ds(0, CHW)],
                    buf_v.at[pl.ds(base + c * CHW, CHW)], sem).wait()

        def chunk_body(c, accs):
            cbase = s + c * CH
            g = jnp.minimum(cbase, N - CH)
            d = cbase - g
            m = jnp.minimum(CH, ln - c * CH)
            rlo = (base // F) + c * CH + d

            def row_body(i, accs2):
                off = i * F
                new = []
                for j in range(nj):
                    x = buf_v[pl.ds(off + j * LANES, LANES)]
                    t = effs[j] * x
                    y = jnp.where(t > 0.0, t, jnp.exp(t) - 1.0)
                    new.append(accs2[j] + y)
                return tuple(new)

            return lax.fori_loop(rlo, rlo + m, row_body, accs)

        zeros = tuple(jnp.zeros((LANES,), jnp.float32) for _ in range(nj))
        accs = lax.fori_loop(0, nc, chunk_body, zeros)
        for j in range(nj):
            row_v[pl.ds(j * LANES, LANES)] = accs[j]
        pltpu.sync_copy(row_v, out_h.at[pl.ds(b * F, F)])

    @pl.when(lo < hi)
    def _():
        read_meta(lo, m0_v)
        fire(m0_v, 0, sem_a)

    npairs = (hi - lo + 1) >> 1

    def pair_body(kk, carry):
        k0 = lo + 2 * kk
        k1 = k0 + 1

        @pl.when(k1 < hi)
        def _():
            read_meta(k1, m1_v)
            fire(m1_v, BASE1, sem_b)

        drain_compute_write(m0_v, 0, sem_a, k0)

        @pl.when(k0 + 2 < hi)
        def _():
            read_meta(k0 + 2, m0_v)
            fire(m0_v, 0, sem_a)

        @pl.when(k1 < hi)
        def _():
            drain_compute_write(m1_v, BASE1, sem_b, k1)

        return carry

    lax.fori_loop(0, npairs, pair_body, 0)


RB = 2048
NB = 92
TC_ROWS = RB * NB
GS = 16
NG = RB // GS


def _tc_body(sstart, send, fseg, nseg, seq_ref, pv_ref, wv_ref, out_ref,
             act_ref):
    j = pl.program_id(0)
    base = j * RB

    @pl.when(j == 0)
    def _():
        out_ref[...] = jnp.zeros_like(out_ref)

    wl0 = wv_ref[0]
    wl1 = wv_ref[1]
    wl2 = wv_ref[2]
    wd0 = wv_ref[3]
    wd1 = wv_ref[4]
    eff = (wd0 * (1.0 + wl0 * pv_ref[0:1, :] + wl1 * pv_ref[1:2, :]
                  + wl2 * pv_ref[2:3, :]) + wd1 * pv_ref[3:4, :])

    t = eff * seq_ref[...]
    act = jnp.where(t > 0.0, t, jnp.exp(t) - 1.0)
    act_ref[...] = act
    gsum = act.reshape(NG, GS, act.shape[1]).sum(axis=1)

    giota = lax.broadcasted_iota(jnp.int32, (NG, 1), 0) * GS + base
    fs = fseg[j]
    ns = nseg[j]

    def seg_body(k, carry):
        s = sstart[k]
        e = send[k]
        full = jnp.logical_and(giota >= s, giota + GS <= e)
        part = jnp.sum(gsum * full.astype(jnp.float32), axis=0,
                       keepdims=True)

        def edge(g):
            rows = act_ref[pl.ds(g * GS, GS), :]
            riota = lax.broadcasted_iota(jnp.int32, (GS, 1), 0) + base + g * GS
            gfull = jnp.logical_and(base + g * GS >= s,
                                    base + g * GS + GS <= e)
            m = ((riota >= s) & (riota < e) & jnp.logical_not(gfull))
            return jnp.sum(rows * m.astype(jnp.float32), axis=0,
                           keepdims=True)

        g0 = jnp.maximum(s - base, 0) // GS
        g1 = jnp.minimum(e - base, RB) - 1
        g1 = jnp.maximum(g1, 0) // GS
        e0 = edge(g0)
        e1 = edge(g1) * jnp.where(g1 != g0, 1.0, 0.0)
        total = part + e0 + e1
        out_ref[pl.ds(k, 1), :] = out_ref[pl.ds(k, 1), :] + total
        return carry

    lax.fori_loop(fs, fs + ns, seg_body, 0)


def kernel(seq, graph_len, prompt1, prompt2, prompt3, w_label, w_dff, w_down):
    N, F = seq.shape
    B = graph_len.shape[0]

    offsets = jnp.cumsum(graph_len)
    starts = jnp.minimum(offsets - graph_len, N).astype(jnp.int32)
    ends = jnp.minimum(offsets, N).astype(jnp.int32)
    lens = ends - starts
    cum = jnp.cumsum(lens)
    totalr = cum[B - 1]

    bs = jnp.searchsorted(ends, TC_ROWS, side="right").astype(jnp.int32)
    t0 = jnp.where(bs > 0, cum[jnp.maximum(bs - 1, 0)], 0)
    rows_sc = totalr - t0
    targets = t0 + (jnp.arange(1, NUM_WORKERS, dtype=jnp.int32) * rows_sc) // NUM_WORKERS
    mids = jnp.clip(jnp.searchsorted(cum, targets, side="left").astype(jnp.int32),
                    bs, B)
    wb = jnp.concatenate([bs[None], mids, jnp.full((1,), B, jnp.int32)])

    blk = jnp.arange(NB, dtype=jnp.int32) * RB
    fseg = jnp.searchsorted(ends, blk, side="right").astype(jnp.int32)
    last = jnp.searchsorted(starts, blk + RB, side="left").astype(jnp.int32)
    nseg = jnp.clip(jnp.minimum(last, bs) - fseg, 0, B)

    meta = jnp.zeros((B * 16 + LANES,), jnp.int32)
    meta = meta.at[0:B * 16:16].set(starts).at[1:B * 16:16].set(lens)
    wlh = jnp.zeros((NUM_WORKERS * 16 + LANES,), jnp.int32)
    idx = jnp.arange(NUM_WORKERS) * 16
    wlh = wlh.at[idx].set(wb[:NUM_WORKERS]).at[idx + 1].set(wb[1:])

    pvec = jnp.concatenate([prompt1.ravel(), prompt2.ravel(),
                            prompt3.ravel(), w_down.ravel()])
    scalars = jnp.concatenate([w_label.ravel(), w_dff.ravel()])
    wsp = jnp.repeat(scalars, LANES)

    pmat = pvec.reshape(4, F)
    wpad = jnp.zeros((8,), jnp.float32).at[:5].set(scalars)
    out_tc = pl.pallas_call(
        _tc_body,
        grid_spec=pltpu.PrefetchScalarGridSpec(
            num_scalar_prefetch=4,
            grid=(NB,),
            in_specs=[
                pl.BlockSpec((RB, F), lambda j, *_: (j, 0)),
                pl.BlockSpec((4, F), lambda j, *_: (0, 0)),
                pl.BlockSpec(memory_space=pltpu.SMEM),
            ],
            out_specs=pl.BlockSpec((B, F), lambda j, *_: (0, 0)),
            scratch_shapes=[pltpu.VMEM((RB, F), jnp.float32)],
        ),
        out_shape=jax.ShapeDtypeStruct((B, F), jnp.float32),
    )(starts, ends, fseg, nseg, seq[:TC_ROWS], pmat, wpad)

    mesh = plsc.VectorSubcoreMesh(core_axis_name="c", subcore_axis_name="s",
                                  num_cores=NUM_CORES,
                                  num_subcores=NUM_SUBCORES)
    body = functools.partial(_sc_body, N, F, B)
    out_flat = pl.kernel(
        body,
        out_type=jax.ShapeDtypeStruct((B * F,), jnp.float32),
        mesh=mesh,
        scratch_types=[
            pltpu.VMEM((2 * SEGROWS * F,), jnp.float32),
            pltpu.VMEM((LANES,), jnp.int32),
            pltpu.VMEM((LANES,), jnp.int32),
            pltpu.VMEM((LANES,), jnp.int32),
            pltpu.VMEM((4 * F,), jnp.float32),
            pltpu.VMEM((5 * LANES,), jnp.float32),
            pltpu.VMEM((F,), jnp.float32),
            pltpu.VMEM((F,), jnp.float32),
            pltpu.SemaphoreType.DMA,
            pltpu.SemaphoreType.DMA,
        ],
    )(seq.reshape(-1), meta, wlh, pvec, wsp)
    out_sc = out_flat.reshape(B, F)

    own_tc = jnp.arange(B, dtype=jnp.int32)[:, None] < bs
    return jnp.where(own_tc, out_tc, out_sc)


# speedup vs baseline: 1.4402x; 1.4402x over previous
import functools

import jax
import jax.numpy as jnp
from jax import lax
from jax.experimental import pallas as pl
from jax.experimental.pallas import tpu as pltpu
from jax.experimental.pallas import tpu_sc as plsc

NUM_CORES = 2
NUM_SUBCORES = 16
NUM_WORKERS = NUM_CORES * NUM_SUBCORES
LANES = 16

GS = 16
RB = 3200
NGB = RB // GS
CHS = 32


def _tc_body(jmax, seq_ref, pv_ref, wv_ref, gact_ref):
    wl0 = wv_ref[0]
    wl1 = wv_ref[1]
    wl2 = wv_ref[2]
    wd0 = wv_ref[3]
    wd1 = wv_ref[4]
    eff = (wd0 * (1.0 + wl0 * pv_ref[0:1, :] + wl1 * pv_ref[1:2, :]
                  + wl2 * pv_ref[2:3, :]) + wd1 * pv_ref[3:4, :])
    t = eff * seq_ref[...]
    act = jnp.where(t > 0.0, t, jnp.exp(t) - 1.0)
    gact_ref[...] = act.reshape(NGB, GS, act.shape[1]).sum(axis=1)


def _sc_body(N, F, B, NGT, seq_h, gact_h, meta_h, wlh_h, pvec_h, wsp_h,
             out_h, gb0_v, lb0_v, hb0_v, gb1_v, lb1_v, hb1_v,
             m0_v, m1_v, wl_v, pv_v, wv_v, eff_v, row_v, sem_a, sem_b):
    nj = F // LANES
    CW = CHS * F

    cid = lax.axis_index("c")
    sid = lax.axis_index("s")
    wid = sid * NUM_CORES + cid

    pltpu.sync_copy(pvec_h, pv_v)
    pltpu.sync_copy(wsp_h, wv_v)
    pltpu.sync_copy(wlh_h.at[pl.ds(wid * 16, LANES)], wl_v)

    wl0 = wv_v[pl.ds(0, LANES)]
    wl1 = wv_v[pl.ds(LANES, LANES)]
    wl2 = wv_v[pl.ds(2 * LANES, LANES)]
    wd0 = wv_v[pl.ds(3 * LANES, LANES)]
    wd1 = wv_v[pl.ds(4 * LANES, LANES)]
    for j in range(nj):
        p1c = pv_v[pl.ds(j * LANES, LANES)]
        p2c = pv_v[pl.ds(F + j * LANES, LANES)]
        p3c = pv_v[pl.ds(2 * F + j * LANES, LANES)]
        wdc = pv_v[pl.ds(3 * F + j * LANES, LANES)]
        eff_v[pl.ds(j * LANES, LANES)] = (
            wd0 * (1.0 + wl0 * p1c + wl1 * p2c + wl2 * p3c) + wd1 * wdc)

    effs = tuple(eff_v[pl.ds(j * LANES, LANES)] for j in range(nj))

    wlv = wl_v[pl.ds(0, LANES)]
    lo = wlv[0]
    hi = wlv[1]

    def read_meta(k, mb):
        pltpu.sync_copy(meta_h.at[pl.ds(k * 16, LANES)], mb)

    def seg_params(mb):
        mv = mb[pl.ds(0, LANES)]
        s = mv[0]
        ln = mv[1]
        e = s + ln
        ga = (s + (GS - 1)) >> 4
        gb = e >> 4
        ng = jnp.maximum(gb - ga, 0)
        locnt = jnp.where(gb > ga, ga * GS - s, ln)
        hicnt = jnp.where(gb > ga, e - gb * GS, 0)
        return s, e, ga, gb, ng, locnt, hicnt

    def fire(mb, gb_v, lb_v, hb_v, sem):
        s, e, ga, gb, ng, locnt, hicnt = seg_params(mb)

        @pl.when(ng > 0)
        def _():
            ag = jnp.minimum(ga, NGT - CHS)
            pltpu.async_copy(gact_h.at[pl.ds(ag * F, CW)], gb_v, sem)

        @pl.when(locnt > 0)
        def _():
            al = jnp.minimum(s, N - CHS)
            pltpu.async_copy(seq_h.at[pl.ds(al * F, CW)], lb_v, sem)

        @pl.when(hicnt > 0)
        def _():
            ah = jnp.minimum(gb * GS, N - CHS)
            pltpu.async_copy(seq_h.at[pl.ds(ah * F, CW)], hb_v, sem)

    def drain_compute_write(mb, gb_v, lb_v, hb_v, sem, b):
        s, e, ga, gb, ng, locnt, hicnt = seg_params(mb)

        @pl.when(ng > 0)
        def _():
            pltpu.make_async_copy(gact_h.at[pl.ds(0, CW)], gb_v, sem).wait()

        @pl.when(locnt > 0)
        def _():
            pltpu.make_async_copy(seq_h.at[pl.ds(0, CW)], lb_v, sem).wait()

        @pl.when(hicnt > 0)
        def _():
            pltpu.make_async_copy(seq_h.at[pl.ds(0, CW)], hb_v, sem).wait()

        zeros = tuple(jnp.zeros((LANES,), jnp.float32) for _ in range(nj))

        dg = ga - jnp.minimum(ga, NGT - CHS)

        def g_body(i, accs):
            off = i * F
            return tuple(accs[j] + gb_v[pl.ds(off + j * LANES, LANES)]
                         for j in range(nj))

        accs = lax.fori_loop(dg, dg + ng, g_body, zeros)

        def edge_body(buf):
            def body(i, accs):
                off = i * F
                new = []
                for j in range(nj):
                    x = buf[pl.ds(off + j * LANES, LANES)]
                    t = effs[j] * x
                    y = jnp.where(t > 0.0, t, jnp.exp(t) - 1.0)
                    new.append(accs[j] + y)
                return tuple(new)
            return body

        dl = s - jnp.minimum(s, N - CHS)
        accs = lax.fori_loop(dl, dl + locnt, edge_body(lb_v), accs)
        dh = gb * GS - jnp.minimum(gb * GS, N - CHS)
        accs = lax.fori_loop(dh, dh + hicnt, edge_body(hb_v), accs)

        for j in range(nj):
            row_v[pl.ds(j * LANES, LANES)] = accs[j]
        pltpu.sync_copy(row_v, out_h.at[pl.ds(b * F, F)])

    @pl.when(lo < hi)
    def _():
        read_meta(lo, m0_v)
        fire(m0_v, gb0_v, lb0_v, hb0_v, sem_a)

    npairs = (hi - lo + 1) >> 1

    def pair_body(kk, carry):
        k0 = lo + 2 * kk
        k1 = k0 + 1

        @pl.when(k1 < hi)
        def _():
            read_meta(k1, m1_v)
            fire(m1_v, gb1_v, lb1_v, hb1_v, sem_b)

        drain_compute_write(m0_v, gb0_v, lb0_v, hb0_v, sem_a, k0)

        @pl.when(k0 + 2 < hi)
        def _():
            read_meta(k0 + 2, m0_v)
            fire(m0_v, gb0_v, lb0_v, hb0_v, sem_a)

        @pl.when(k1 < hi)
        def _():
            drain_compute_write(m1_v, gb1_v, lb1_v, hb1_v, sem_b, k1)

        return carry

    lax.fori_loop(0, npairs, pair_body, 0)


def kernel(seq, graph_len, prompt1, prompt2, prompt3, w_label, w_dff, w_down):
    N, F = seq.shape
    B = graph_len.shape[0]
    NB = N // RB
    NGT = N // GS

    offsets = jnp.cumsum(graph_len)
    starts = jnp.minimum(offsets - graph_len, N).astype(jnp.int32)
    ends = jnp.minimum(offsets, N).astype(jnp.int32)
    lens = ends - starts
    cum = jnp.cumsum(lens)
    totalr = cum[B - 1]
    targets = (jnp.arange(1, NUM_WORKERS, dtype=jnp.int32) * totalr) // NUM_WORKERS
    mids = jnp.searchsorted(cum, targets, side="left").astype(jnp.int32)
    wb = jnp.concatenate([jnp.zeros((1,), jnp.int32), mids,
                          jnp.full((1,), B, jnp.int32)])

    meta = jnp.zeros((B * 16 + LANES,), jnp.int32)
    meta = meta.at[0:B * 16:16].set(starts).at[1:B * 16:16].set(lens)
    wlh = jnp.zeros((NUM_WORKERS * 16 + LANES,), jnp.int32)
    idx = jnp.arange(NUM_WORKERS) * 16
    wlh = wlh.at[idx].set(wb[:NUM_WORKERS]).at[idx + 1].set(wb[1:])

    pvec = jnp.concatenate([prompt1.ravel(), prompt2.ravel(),
                            prompt3.ravel(), w_down.ravel()])
    scalars = jnp.concatenate([w_label.ravel(), w_dff.ravel()])
    wsp = jnp.repeat(scalars, LANES)

    pmat = pvec.reshape(4, F)
    wpad = jnp.zeros((8,), jnp.float32).at[:5].set(scalars)
    jmax = jnp.maximum((totalr + RB - 1) // RB, 1).astype(jnp.int32)[None]
    gact = pl.pallas_call(
        _tc_body,
        grid_spec=pltpu.PrefetchScalarGridSpec(
            num_scalar_prefetch=1,
            grid=(NB,),
            in_specs=[
                pl.BlockSpec((RB, F), lambda j, jm: (jnp.minimum(j, jm[0] - 1), 0)),
                pl.BlockSpec((4, F), lambda j, jm: (0, 0)),
                pl.BlockSpec(memory_space=pltpu.SMEM),
            ],
            out_specs=pl.BlockSpec(
                (NGB, F), lambda j, jm: (jnp.minimum(j, jm[0] - 1), 0)),
        ),
        out_shape=jax.ShapeDtypeStruct((NGT, F), jnp.float32),
    )(jmax, seq, pmat, wpad)

    mesh = plsc.VectorSubcoreMesh(core_axis_name="c", subcore_axis_name="s",
                                  num_cores=NUM_CORES,
                                  num_subcores=NUM_SUBCORES)
    body = functools.partial(_sc_body, N, F, B, NGT)
    out_flat = pl.kernel(
        body,
        out_type=jax.ShapeDtypeStruct((B * F,), jnp.float32),
        mesh=mesh,
        scratch_types=[
            pltpu.VMEM((CHS * F,), jnp.float32),
            pltpu.VMEM((CHS * F,), jnp.float32),
            pltpu.VMEM((CHS * F,), jnp.float32),
            pltpu.VMEM((CHS * F,), jnp.float32),
            pltpu.VMEM((CHS * F,), jnp.float32),
            pltpu.VMEM((CHS * F,), jnp.float32),
            pltpu.VMEM((LANES,), jnp.int32),
            pltpu.VMEM((LANES,), jnp.int32),
            pltpu.VMEM((LANES,), jnp.int32),
            pltpu.VMEM((4 * F,), jnp.float32),
            pltpu.VMEM((5 * LANES,), jnp.float32),
            pltpu.VMEM((F,), jnp.float32),
            pltpu.VMEM((F,), jnp.float32),
            pltpu.SemaphoreType.DMA,
            pltpu.SemaphoreType.DMA,
        ],
    )(seq.reshape(-1), gact.reshape(-1), meta, wlh, pvec, wsp)
    return out_flat.reshape(B, F)


# speedup vs baseline: 1.7506x; 1.2156x over previous
import functools

import jax
import jax.numpy as jnp
from jax import lax
from jax.experimental import pallas as pl
from jax.experimental.pallas import tpu as pltpu
from jax.experimental.pallas import tpu_sc as plsc

NUM_CORES = 2
NUM_SUBCORES = 16
NUM_WORKERS = NUM_CORES * NUM_SUBCORES
LANES = 16

GS = 16
RB = 6400
NGB = RB // GS
CHS = 32
CHE = 16


def _tc_body(jmax, seq_ref, pv_ref, wv_ref, gact_ref):
    wl0 = wv_ref[0]
    wl1 = wv_ref[1]
    wl2 = wv_ref[2]
    wd0 = wv_ref[3]
    wd1 = wv_ref[4]
    eff = (wd0 * (1.0 + wl0 * pv_ref[0:1, :] + wl1 * pv_ref[1:2, :]
                  + wl2 * pv_ref[2:3, :]) + wd1 * pv_ref[3:4, :])
    t = eff * seq_ref[...]
    act = jnp.where(t > 0.0, t, jnp.exp(t) - 1.0)
    gact_ref[...] = act.reshape(NGB, GS, act.shape[1]).sum(axis=1)


def _sc_body(N, F, B, NGT, seq_h, gact_h, meta_h, wlh_h, pvec_h, wsp_h,
             out_h, gb0_v, lb0_v, hb0_v, gb1_v, lb1_v, hb1_v,
             m0_v, m1_v, wl_v, pv_v, wv_v, eff_v, row_v, sem_a, sem_b):
    nj = F // LANES
    CW = CHS * F

    cid = lax.axis_index("c")
    sid = lax.axis_index("s")
    wid = sid * NUM_CORES + cid

    pltpu.sync_copy(pvec_h, pv_v)
    pltpu.sync_copy(wsp_h, wv_v)
    pltpu.sync_copy(wlh_h.at[pl.ds(wid * 16, LANES)], wl_v)

    wl0 = wv_v[pl.ds(0, LANES)]
    wl1 = wv_v[pl.ds(LANES, LANES)]
    wl2 = wv_v[pl.ds(2 * LANES, LANES)]
    wd0 = wv_v[pl.ds(3 * LANES, LANES)]
    wd1 = wv_v[pl.ds(4 * LANES, LANES)]
    for j in range(nj):
        p1c = pv_v[pl.ds(j * LANES, LANES)]
        p2c = pv_v[pl.ds(F + j * LANES, LANES)]
        p3c = pv_v[pl.ds(2 * F + j * LANES, LANES)]
        wdc = pv_v[pl.ds(3 * F + j * LANES, LANES)]
        eff_v[pl.ds(j * LANES, LANES)] = (
            wd0 * (1.0 + wl0 * p1c + wl1 * p2c + wl2 * p3c) + wd1 * wdc)

    effs = tuple(eff_v[pl.ds(j * LANES, LANES)] for j in range(nj))

    wlv = wl_v[pl.ds(0, LANES)]
    lo = wlv[0]
    hi = wlv[1]

    def read_meta(k, mb):
        pltpu.sync_copy(meta_h.at[pl.ds(k * 16, LANES)], mb)

    def seg_params(mb):
        mv = mb[pl.ds(0, LANES)]
        s = mv[0]
        ln = mv[1]
        e = s + ln
        ga = (s + (GS - 1)) >> 4
        gb = e >> 4
        ng = jnp.maximum(gb - ga, 0)
        locnt = jnp.where(gb > ga, ga * GS - s, ln)
        hicnt = jnp.where(gb > ga, e - gb * GS, 0)
        return s, e, ga, gb, ng, locnt, hicnt

    def fire(mb, gb_v, lb_v, hb_v, sem):
        s, e, ga, gb, ng, locnt, hicnt = seg_params(mb)

        @pl.when(ng > 0)
        def _():
            ag = jnp.minimum(ga, NGT - CHS)
            pltpu.async_copy(gact_h.at[pl.ds(ag * F, CW)], gb_v, sem)

        @pl.when(locnt > 0)
        def _():
            al = jnp.minimum(s, N - CHS)
            pltpu.async_copy(seq_h.at[pl.ds(al * F, CW)], lb_v, sem)

        @pl.when(hicnt > 0)
        def _():
            ah = jnp.minimum(gb * GS, N - CHE)
            pltpu.async_copy(seq_h.at[pl.ds(ah * F, CHE * F)], hb_v, sem)

    def drain_compute_write(mb, gb_v, lb_v, hb_v, sem, b):
        s, e, ga, gb, ng, locnt, hicnt = seg_params(mb)

        @pl.when(ng > 0)
        def _():
            pltpu.make_async_copy(gact_h.at[pl.ds(0, CW)], gb_v, sem).wait()

        @pl.when(locnt > 0)
        def _():
            pltpu.make_async_copy(seq_h.at[pl.ds(0, CW)], lb_v, sem).wait()

        @pl.when(hicnt > 0)
        def _():
            pltpu.make_async_copy(seq_h.at[pl.ds(0, CHE * F)], hb_v,
                                  sem).wait()

        zeros = tuple(jnp.zeros((LANES,), jnp.float32) for _ in range(nj))

        dg = ga - jnp.minimum(ga, NGT - CHS)

        def g_body(i, accs):
            off = i * F
            return tuple(accs[j] + gb_v[pl.ds(off + j * LANES, LANES)]
                         for j in range(nj))

        accs = lax.fori_loop(dg, dg + ng, g_body, zeros)

        def edge_body(buf):
            def body(i, accs):
                off = i * F
                new = []
                for j in range(nj):
                    x = buf[pl.ds(off + j * LANES, LANES)]
                    t = effs[j] * x
                    y = jnp.where(t > 0.0, t, jnp.exp(t) - 1.0)
                    new.append(accs[j] + y)
                return tuple(new)
            return body

        dl = s - jnp.minimum(s, N - CHS)
        accs = lax.fori_loop(dl, dl + locnt, edge_body(lb_v), accs)
        dh = gb * GS - jnp.minimum(gb * GS, N - CHE)
        accs = lax.fori_loop(dh, dh + hicnt, edge_body(hb_v), accs)

        for j in range(nj):
            row_v[pl.ds(j * LANES, LANES)] = accs[j]
        pltpu.sync_copy(row_v, out_h.at[pl.ds(b * F, F)])

    @pl.when(lo < hi)
    def _():
        read_meta(lo, m0_v)
        fire(m0_v, gb0_v, lb0_v, hb0_v, sem_a)

    npairs = (hi - lo + 1) >> 1

    def pair_body(kk, carry):
        k0 = lo + 2 * kk
        k1 = k0 + 1

        @pl.when(k1 < hi)
        def _():
            read_meta(k1, m1_v)
            fire(m1_v, gb1_v, lb1_v, hb1_v, sem_b)

        drain_compute_write(m0_v, gb0_v, lb0_v, hb0_v, sem_a, k0)

        @pl.when(k0 + 2 < hi)
        def _():
            read_meta(k0 + 2, m0_v)
            fire(m0_v, gb0_v, lb0_v, hb0_v, sem_a)

        @pl.when(k1 < hi)
        def _():
            drain_compute_write(m1_v, gb1_v, lb1_v, hb1_v, sem_b, k1)

        return carry

    lax.fori_loop(0, npairs, pair_body, 0)


def kernel(seq, graph_len, prompt1, prompt2, prompt3, w_label, w_dff, w_down):
    N, F = seq.shape
    B = graph_len.shape[0]
    NB = N // RB
    NGT = N // GS

    offsets = jnp.cumsum(graph_len)
    starts = jnp.minimum(offsets - graph_len, N).astype(jnp.int32)
    ends = jnp.minimum(offsets, N).astype(jnp.int32)
    lens = ends - starts
    totalr = ends[B - 1]
    targets = (jnp.arange(1, NUM_WORKERS, dtype=jnp.int32) * totalr) // NUM_WORKERS
    mids = jnp.searchsorted(ends, targets, side="left").astype(jnp.int32)
    wb = jnp.concatenate([jnp.zeros((1,), jnp.int32), mids,
                          jnp.full((1,), B, jnp.int32)])

    meta = jnp.concatenate(
        [jnp.stack([starts, lens], axis=1),
         jnp.zeros((B, 14), jnp.int32)], axis=1).reshape(-1)
    meta = jnp.concatenate([meta, jnp.zeros((LANES,), jnp.int32)])
    wlh = jnp.concatenate(
        [jnp.stack([wb[:NUM_WORKERS], wb[1:]], axis=1),
         jnp.zeros((NUM_WORKERS, 14), jnp.int32)], axis=1).reshape(-1)
    wlh = jnp.concatenate([wlh, jnp.zeros((LANES,), jnp.int32)])

    pvec = jnp.concatenate([prompt1.ravel(), prompt2.ravel(),
                            prompt3.ravel(), w_down.ravel()])
    scalars = jnp.concatenate([w_label.ravel(), w_dff.ravel()])
    wsp = jnp.repeat(scalars, LANES)

    pmat = pvec.reshape(4, F)
    wpad = jnp.zeros((8,), jnp.float32).at[:5].set(scalars)
    jmax = jnp.maximum((totalr + RB - 1) // RB, 1).astype(jnp.int32)[None]
    gact = pl.pallas_call(
        _tc_body,
        grid_spec=pltpu.PrefetchScalarGridSpec(
            num_scalar_prefetch=1,
            grid=(NB,),
            in_specs=[
                pl.BlockSpec((RB, F), lambda j, jm: (jnp.minimum(j, jm[0] - 1), 0)),
                pl.BlockSpec((4, F), lambda j, jm: (0, 0)),
                pl.BlockSpec(memory_space=pltpu.SMEM),
            ],
            out_specs=pl.BlockSpec(
                (NGB, F), lambda j, jm: (jnp.minimum(j, jm[0] - 1), 0)),
        ),
        out_shape=jax.ShapeDtypeStruct((NGT, F), jnp.float32),
    )(jmax, seq, pmat, wpad)

    mesh = plsc.VectorSubcoreMesh(core_axis_name="c", subcore_axis_name="s",
                                  num_cores=NUM_CORES,
                                  num_subcores=NUM_SUBCORES)
    body = functools.partial(_sc_body, N, F, B, NGT)
    out_flat = pl.kernel(
        body,
        out_type=jax.ShapeDtypeStruct((B * F,), jnp.float32),
        mesh=mesh,
        scratch_types=[
            pltpu.VMEM((CHS * F,), jnp.float32),
            pltpu.VMEM((CHS * F,), jnp.float32),
            pltpu.VMEM((CHE * F,), jnp.float32),
            pltpu.VMEM((CHS * F,), jnp.float32),
            pltpu.VMEM((CHS * F,), jnp.float32),
            pltpu.VMEM((CHE * F,), jnp.float32),
            pltpu.VMEM((LANES,), jnp.int32),
            pltpu.VMEM((LANES,), jnp.int32),
            pltpu.VMEM((LANES,), jnp.int32),
            pltpu.VMEM((4 * F,), jnp.float32),
            pltpu.VMEM((5 * LANES,), jnp.float32),
            pltpu.VMEM((F,), jnp.float32),
            pltpu.VMEM((F,), jnp.float32),
            pltpu.SemaphoreType.DMA,
            pltpu.SemaphoreType.DMA,
        ],
    )(seq.reshape(-1), gact.reshape(-1), meta, wlh, pvec, wsp)
    return out_flat.reshape(B, F)


# speedup vs baseline: 2.0279x; 1.1584x over previous
import functools

import jax
import jax.numpy as jnp
from jax import lax
from jax.experimental import pallas as pl
from jax.experimental.pallas import tpu as pltpu
from jax.experimental.pallas import tpu_sc as plsc

NUM_CORES = 2
NUM_SUBCORES = 16
NUM_WORKERS = NUM_CORES * NUM_SUBCORES
LANES = 16

GS = 16
RB = 6400
NGB = RB // GS
CHS = 32
CHE = 16
BP = 1024


def _bk_body(B, N, gl_ref, p1_ref, p2_ref, p3_ref, wdn_ref, wlab_ref,
             wdff_ref, off_ref, wb_ref, jm_ref, eff_ref):
    gl8 = gl_ref[...]
    glf = gl8.astype(jnp.float32)

    io_r = lax.broadcasted_iota(jnp.int32, (128, 128), 0)
    io_c = lax.broadcasted_iota(jnp.int32, (128, 128), 1)
    tri = (io_r <= io_c).astype(jnp.float32)
    s1 = jnp.dot(glf, tri,
                 precision=lax.Precision.HIGHEST)
    rowtot = s1[:, 127:128]

    def shift(x, k):
        return jnp.concatenate(
            [jnp.zeros((k, 1), jnp.float32), x[:8 - k, :]], axis=0)

    s = rowtot
    s = s + shift(s, 1)
    s = s + shift(s, 2)
    s = s + shift(s, 4)
    rowpre = s - rowtot

    off_i = (s1 + rowpre).astype(jnp.int32)
    off_ref[...] = off_i

    ends8 = jnp.minimum(off_i, N)
    totalr = jnp.max(ends8)

    lane48 = lax.broadcasted_iota(jnp.int32, (1, 48), 1)
    acc = jnp.where(lane48 == NUM_WORKERS, B, 0)
    for w in range(1, NUM_WORKERS):
        tw = (w * totalr) >> 5
        cnt = jnp.sum((ends8 < tw).astype(jnp.int32))
        acc = acc + jnp.where(lane48 == w, cnt, 0)
    wb_ref[...] = acc

    jm = jnp.maximum((totalr + RB - 1) // RB, 1)
    jm_ref[...] = jnp.reshape(jm, (1, 1))

    wl0 = wlab_ref[0]
    wl1 = wlab_ref[1]
    wl2 = wlab_ref[2]
    wd0 = wdff_ref[0]
    wd1 = wdff_ref[1]
    eff_ref[...] = (wd0 * (1.0 + wl0 * p1_ref[...] + wl1 * p2_ref[...]
                           + wl2 * p3_ref[...]) + wd1 * wdn_ref[...])


def _tc_body(jmax, seq_ref, eff_ref, gact_ref):
    t = eff_ref[...] * seq_ref[...]
    act = jnp.where(t > 0.0, t, jnp.exp(t) - 1.0)
    gact_ref[...] = act.reshape(NGB, GS, act.shape[1]).sum(axis=1)


def _sc_body(N, F, B, NGT, seq_h, gact_h, off_h, wb_h, eff_h,
             out_h, gb0_v, lb0_v, hb0_v, gb1_v, lb1_v, hb1_v,
             off_v, wbv_v, eff_v, row_v, sem_a, sem_b):
    nj = F // LANES
    CW = CHS * F

    cid = lax.axis_index("c")
    sid = lax.axis_index("s")
    wid = sid * NUM_CORES + cid

    pltpu.sync_copy(off_h, off_v)
    pltpu.sync_copy(wb_h, wbv_v)
    pltpu.sync_copy(eff_h, eff_v)

    effs = tuple(eff_v[pl.ds(j * LANES, LANES)] for j in range(nj))
    io16 = jnp.arange(LANES, dtype=jnp.int32)

    def pick(v, i):
        r = v[8]
        for q in range(7, -1, -1):
            r = jnp.where(i == q, v[q], r)
        return r

    wa = (wid >> 3) << 3
    vw = wbv_v[pl.ds(wa, LANES)]
    lo = pick(vw, wid - wa)
    hi = pick(vw, wid + 1 - wa)

    def seg_params(k):
        km = jnp.maximum(k - 1, 0)
        a = (km >> 3) << 3
        v = off_v[pl.ds(a, LANES)]
        sp = jnp.where(k == 0, 0, pick(v, km - a))
        s = jnp.minimum(sp, N)
        e = jnp.minimum(pick(v, k - a), N)
        ln = e - s
        ga = (s + (GS - 1)) >> 4
        gb = e >> 4
        ng = jnp.maximum(gb - ga, 0)
        locnt = jnp.where(gb > ga, ga * GS - s, ln)
        hicnt = jnp.where(gb > ga, e - gb * GS, 0)
        return s, e, ga, gb, ng, locnt, hicnt

    def fire(k, gb_v, lb_v, hb_v, sem):
        s, e, ga, gb, ng, locnt, hicnt = seg_params(k)

        @pl.when(ng > 0)
        def _():
            ag = jnp.minimum(ga, NGT - CHS)
            pltpu.async_copy(gact_h.at[pl.ds(ag * F, CW)], gb_v, sem)

        @pl.when(locnt > 0)
        def _():
            al = jnp.minimum(s, N - CHS)
            pltpu.async_copy(seq_h.at[pl.ds(al * F, CW)], lb_v, sem)

        @pl.when(hicnt > 0)
        def _():
            ah = jnp.minimum(gb * GS, N - CHE)
            pltpu.async_copy(seq_h.at[pl.ds(ah * F, CHE * F)], hb_v, sem)

    def drain_compute_write(k, gb_v, lb_v, hb_v, sem, b):
        s, e, ga, gb, ng, locnt, hicnt = seg_params(k)

        @pl.when(ng > 0)
        def _():
            pltpu.make_async_copy(gact_h.at[pl.ds(0, CW)], gb_v, sem).wait()

        @pl.when(locnt > 0)
        def _():
            pltpu.make_async_copy(seq_h.at[pl.ds(0, CW)], lb_v, sem).wait()

        @pl.when(hicnt > 0)
        def _():
            pltpu.make_async_copy(seq_h.at[pl.ds(0, CHE * F)], hb_v,
                                  sem).wait()

        zeros = tuple(jnp.zeros((LANES,), jnp.float32) for _ in range(nj))

        dg = ga - jnp.minimum(ga, NGT - CHS)

        def g_body(i, accs):
            off = i * F
            return tuple(accs[j] + gb_v[pl.ds(off + j * LANES, LANES)]
                         for j in range(nj))

        accs = lax.fori_loop(dg, dg + ng, g_body, zeros)

        def edge_body(buf):
            def body(i, accs):
                off = i * F
                new = []
                for j in range(nj):
                    x = buf[pl.ds(off + j * LANES, LANES)]
                    t = effs[j] * x
                    y = jnp.where(t > 0.0, t, jnp.exp(t) - 1.0)
                    new.append(accs[j] + y)
                return tuple(new)
            return body

        dl = s - jnp.minimum(s, N - CHS)
        accs = lax.fori_loop(dl, dl + locnt, edge_body(lb_v), accs)
        dh = gb * GS - jnp.minimum(gb * GS, N - CHE)
        accs = lax.fori_loop(dh, dh + hicnt, edge_body(hb_v), accs)

        for j in range(nj):
            row_v[pl.ds(j * LANES, LANES)] = accs[j]
        pltpu.sync_copy(row_v, out_h.at[pl.ds(b * F, F)])

    @pl.when(lo < hi)
    def _():
        fire(lo, gb0_v, lb0_v, hb0_v, sem_a)

    npairs = (hi - lo + 1) >> 1

    def pair_body(kk, carry):
        k0 = lo + 2 * kk
        k1 = k0 + 1

        @pl.when(k1 < hi)
        def _():
            fire(k1, gb1_v, lb1_v, hb1_v, sem_b)

        drain_compute_write(k0, gb0_v, lb0_v, hb0_v, sem_a, k0)

        @pl.when(k0 + 2 < hi)
        def _():
            fire(k0 + 2, gb0_v, lb0_v, hb0_v, sem_a)

        @pl.when(k1 < hi)
        def _():
            drain_compute_write(k1, gb1_v, lb1_v, hb1_v, sem_b, k1)

        return carry

    lax.fori_loop(0, npairs, pair_body, 0)


def kernel(seq, graph_len, prompt1, prompt2, prompt3, w_label, w_dff, w_down):
    N, F = seq.shape
    B = graph_len.shape[0]
    NB = N // RB
    NGT = N // GS

    gl8 = jnp.concatenate(
        [graph_len.astype(jnp.int32),
         jnp.zeros((BP - B,), jnp.int32)]).reshape(8, BP // 8)

    off8, wb48, jm, eff = pl.pallas_call(
        functools.partial(_bk_body, B, N),
        grid=(1,),
        in_specs=[
            pl.BlockSpec((8, BP // 8), lambda i: (0, 0)),
            pl.BlockSpec((1, F), lambda i: (0, 0)),
            pl.BlockSpec((1, F), lambda i: (0, 0)),
            pl.BlockSpec((1, F), lambda i: (0, 0)),
            pl.BlockSpec((1, F), lambda i: (0, 0)),
            pl.BlockSpec(memory_space=pltpu.SMEM),
            pl.BlockSpec(memory_space=pltpu.SMEM),
        ],
        out_specs=[
            pl.BlockSpec((8, BP // 8), lambda i: (0, 0)),
            pl.BlockSpec((1, 48), lambda i: (0, 0)),
            pl.BlockSpec((1, 1), lambda i: (0, 0)),
            pl.BlockSpec((1, F), lambda i: (0, 0)),
        ],
        out_shape=[
            jax.ShapeDtypeStruct((8, BP // 8), jnp.int32),
            jax.ShapeDtypeStruct((1, 48), jnp.int32),
            jax.ShapeDtypeStruct((1, 1), jnp.int32),
            jax.ShapeDtypeStruct((1, F), jnp.float32),
        ],
    )(gl8, prompt1, prompt2, prompt3, w_down,
      w_label.reshape(-1), w_dff.reshape(-1))

    gact = pl.pallas_call(
        _tc_body,
        grid_spec=pltpu.PrefetchScalarGridSpec(
            num_scalar_prefetch=1,
            grid=(NB,),
            in_specs=[
                pl.BlockSpec((RB, F), lambda j, jm: (jnp.minimum(j, jm[0] - 1), 0)),
                pl.BlockSpec((1, F), lambda j, jm: (0, 0)),
            ],
            out_specs=pl.BlockSpec(
                (NGB, F), lambda j, jm: (jnp.minimum(j, jm[0] - 1), 0)),
        ),
        out_shape=jax.ShapeDtypeStruct((NGT, F), jnp.float32),
    )(jm.reshape(1), seq, eff)

    mesh = plsc.VectorSubcoreMesh(core_axis_name="c", subcore_axis_name="s",
                                  num_cores=NUM_CORES,
                                  num_subcores=NUM_SUBCORES)
    body = functools.partial(_sc_body, N, F, B, NGT)
    out_flat = pl.kernel(
        body,
        out_type=jax.ShapeDtypeStruct((B * F,), jnp.float32),
        mesh=mesh,
        scratch_types=[
            pltpu.VMEM((CHS * F,), jnp.float32),
            pltpu.VMEM((CHS * F,), jnp.float32),
            pltpu.VMEM((CHE * F,), jnp.float32),
            pltpu.VMEM((CHS * F,), jnp.float32),
            pltpu.VMEM((CHS * F,), jnp.float32),
            pltpu.VMEM((CHE * F,), jnp.float32),
            pltpu.VMEM((BP,), jnp.int32),
            pltpu.VMEM((48,), jnp.int32),
            pltpu.VMEM((F,), jnp.float32),
            pltpu.VMEM((F,), jnp.float32),
            pltpu.SemaphoreType.DMA,
            pltpu.SemaphoreType.DMA,
        ],
    )(seq.reshape(-1), gact.reshape(-1), off8.reshape(-1), wb48.reshape(-1),
      eff.reshape(-1))
    return out_flat.reshape(B, F)
